# Initial kernel scaffold; baseline (speedup 1.0000x reference)
import jax
import jax.numpy as jnp
from jax.experimental import pallas as pl


def kernel(atomic_numbers, neighbors, neighbor_mask, distances, unit_vecs, params):
    raise NotImplementedError("write your pallas kernel here")


# speedup vs baseline: 4.8354x; 4.8354x over previous
import functools

import jax
import jax.numpy as jnp
from jax import lax
from jax.experimental import pallas as pl
from jax.experimental.pallas import tpu as pltpu
from jax.experimental.pallas import tpu_sc as plsc

GAUSS_END = 5.5

_SC_CORES = 2
_SC_SUBCORES = 16
_NW = _SC_CORES * _SC_SUBCORES
_CHUNK = 128


def _sc_gather(table, idx):
    bsz = idx.shape[0]
    dim = table.shape[1]
    assert bsz % (8 * _NW) == 0, bsz
    bpw = bsz // _NW
    full = bpw // _CHUNK
    rem = bpw % _CHUNK
    assert rem % 8 == 0, rem
    mesh = plsc.VectorSubcoreMesh(core_axis_name="c", subcore_axis_name="s")

    @functools.partial(
        pl.kernel,
        out_type=jax.ShapeDtypeStruct((bsz, dim), table.dtype),
        mesh=mesh,
        scratch_types=[
            pltpu.VMEM((_CHUNK,), jnp.int32),
            pltpu.VMEM((_CHUNK, dim), table.dtype),
            pltpu.SemaphoreType.DMA,
        ],
    )
    def gather_kernel(table_hbm, idx_hbm, out_hbm, idx_v, rows_v, sem):
        wid = lax.axis_index("s") * _SC_CORES + lax.axis_index("c")
        base = wid * bpw

        def chunk(off, nrows):
            iv = idx_v.at[pl.ds(0, nrows)] if nrows != _CHUNK else idx_v
            rv = rows_v.at[pl.ds(0, nrows)] if nrows != _CHUNK else rows_v
            pltpu.sync_copy(idx_hbm.at[pl.ds(off, nrows)], iv)
            pltpu.async_copy(table_hbm.at[iv], rv, sem).wait()
            pltpu.sync_copy(rv, out_hbm.at[pl.ds(off, nrows)])

        if full:
            def body(g, carry):
                chunk(base + g * _CHUNK, _CHUNK)
                return carry
            lax.fori_loop(0, full, body, 0)
        if rem:
            chunk(base + full * _CHUNK, rem)

    return gather_kernel(table, idx)


def _ssp(x):
    return jnp.maximum(x, 0.0) + jnp.log1p(jnp.exp(-jnp.abs(x))) - 0.6931471805599453


def _sigmoid(x):
    return 1.0 / (1.0 + jnp.exp(-x))


_T = 200


def _mp_layer(h, nbr_h, e_in, mask_col, w, b, fn, nbr):
    at = h.shape[0]
    te = _T * nbr
    first = e_in.shape[1] == 1
    grid = at // _T

    def body(h_ref, nbr_ref, e_ref, m_ref, w_ref, b_ref, ho_ref, eo_ref):
        i = pl.program_id(0)
        h_tile = h_ref[pl.ds(i * _T, _T), :]
        if first:
            d = e_ref[...]
            width = GAUSS_END / (fn - 1)
            offs = lax.broadcasted_iota(
                jnp.int32, (1, fn), 1).astype(jnp.float32) * width
            e = jnp.exp(-0.5 * ((d - offs) / width) ** 2)
        else:
            e = e_ref[...]
        nbrv = nbr_ref[...]
        z_hi = jnp.dot(h_tile.astype(jnp.bfloat16), w_ref[0:fn, :],
                       preferred_element_type=jnp.float32)
        z = jnp.dot(nbrv.astype(jnp.bfloat16), w_ref[fn:2 * fn, :],
                    preferred_element_type=jnp.float32)
        z = z + jnp.dot(e.astype(jnp.bfloat16), w_ref[2 * fn:3 * fn, :],
                        preferred_element_type=jnp.float32)
        z = z + jnp.broadcast_to(z_hi[:, None, :], (_T, nbr, 4 * fn)
                                 ).reshape(te, 4 * fn)
        z = z + b_ref[...]
        m = m_ref[...]
        gate_n = _sigmoid(z[:, 0:fn])
        core_n = _ssp(z[:, fn:2 * fn])
        gate_e = _sigmoid(z[:, 2 * fn:3 * fn])
        core_e = _ssp(z[:, 3 * fn:4 * fn])
        pn = gate_n * core_n * m
        ho_ref[...] = h_tile + pn.reshape(_T, nbr, fn).sum(axis=1)
        eo_ref[...] = e + gate_e * core_e * m

    return pl.pallas_call(
        body,
        grid=(grid,),
        in_specs=[
            pl.BlockSpec((at, fn), lambda i: (0, 0)),
            pl.BlockSpec((te, fn), lambda i: (i, 0)),
            pl.BlockSpec((te, e_in.shape[1]), lambda i: (i, 0)),
            pl.BlockSpec((te, 1), lambda i: (i, 0)),
            pl.BlockSpec(w.shape, lambda i: (0, 0)),
            pl.BlockSpec(b.shape, lambda i: (0, 0)),
        ],
        out_specs=[
            pl.BlockSpec((_T, fn), lambda i: (i, 0)),
            pl.BlockSpec((te, fn), lambda i: (i, 0)),
        ],
        out_shape=[
            jax.ShapeDtypeStruct((at, fn), jnp.float32),
            jax.ShapeDtypeStruct((at * nbr, fn), jnp.float32),
        ],
    )(h, nbr_h, e_in, mask_col, w, b)


def _last_layer(h, nbr_h, e_in, mask_col, uv, w, b, w1, b1, w2, b2, fn, nbr):
    at = h.shape[0]
    te = _T * nbr
    grid = at // _T
    fh = w1.shape[1]

    def body(h_ref, nbr_ref, e_ref, m_ref, uv_ref, w_ref, b_ref,
             w1_ref, b1_ref, w2_ref, b2_ref, f_ref):
        i = pl.program_id(0)
        h_tile = h_ref[pl.ds(i * _T, _T), :]
        e = e_ref[...]
        nbrv = nbr_ref[...]
        z_hi = jnp.dot(h_tile.astype(jnp.bfloat16), w_ref[0:fn, :],
                       preferred_element_type=jnp.float32)
        z = jnp.dot(nbrv.astype(jnp.bfloat16), w_ref[fn:2 * fn, :],
                    preferred_element_type=jnp.float32)
        z = z + jnp.dot(e.astype(jnp.bfloat16), w_ref[2 * fn:3 * fn, :],
                        preferred_element_type=jnp.float32)
        z = z + jnp.broadcast_to(z_hi[:, None, :], (_T, nbr, 2 * fn)
                                 ).reshape(te, 2 * fn)
        z = z + b_ref[...]
        m = m_ref[...]
        gate_e = _sigmoid(z[:, 0:fn])
        core_e = _ssp(z[:, fn:2 * fn])
        e3 = e + gate_e * core_e * m
        x = _ssp(jnp.dot(e3.astype(jnp.bfloat16), w1_ref[...],
                         preferred_element_type=jnp.float32) + b1_ref[...])
        s = jnp.dot(x.astype(jnp.bfloat16), w2_ref[...],
                    preferred_element_type=jnp.float32) + b2_ref[...]
        f = s * uv_ref[...] * m
        f_ref[...] = f.reshape(_T, nbr, 3).sum(axis=1)

    return pl.pallas_call(
        body,
        grid=(grid,),
        in_specs=[
            pl.BlockSpec((at, fn), lambda i: (0, 0)),
            pl.BlockSpec((te, fn), lambda i: (i, 0)),
            pl.BlockSpec((te, fn), lambda i: (i, 0)),
            pl.BlockSpec((te, 1), lambda i: (i, 0)),
            pl.BlockSpec((te, 3), lambda i: (i, 0)),
            pl.BlockSpec(w.shape, lambda i: (0, 0)),
            pl.BlockSpec(b.shape, lambda i: (0, 0)),
            pl.BlockSpec(w1.shape, lambda i: (0, 0)),
            pl.BlockSpec(b1.shape, lambda i: (0, 0)),
            pl.BlockSpec(w2.shape, lambda i: (0, 0)),
            pl.BlockSpec(b2.shape, lambda i: (0, 0)),
        ],
        out_specs=[pl.BlockSpec((_T, 3), lambda i: (i, 0))],
        out_shape=[jax.ShapeDtypeStruct((at, 3), jnp.float32)],
    )(h, nbr_h, e_in, mask_col, uv, w, b, w1, b1, w2, b2)[0]


def kernel(atomic_numbers, neighbors, neighbor_mask, distances, unit_vecs, params):
    b, at, nbr = neighbors.shape
    fn = params['emb_table'].shape[1]
    atn = at * nbr

    an_flat = atomic_numbers.reshape(-1).astype(jnp.int32)
    pad = (-an_flat.shape[0]) % (8 * _NW)
    an_pad = jnp.pad(an_flat, (0, pad))
    nbr_flat = neighbors.reshape(-1).astype(jnp.int32)
    mask_col = neighbor_mask.reshape(atn, 1)
    dist_col = distances.reshape(atn, 1)
    uv_flat = unit_vecs.reshape(atn, 3)

    layers = params['layers']
    ws, bs = [], []
    for li in range(len(layers)):
        lyr = layers[li]
        if li + 1 < len(layers):
            w = jnp.concatenate(
                [lyr['Wf_n'], lyr['Ws_n'], lyr['Wf_e'], lyr['Ws_e']], axis=1)
            bias = jnp.concatenate(
                [lyr['bf_n'], lyr['bs_n'], lyr['bf_e'], lyr['bs_e']])
        else:
            w = jnp.concatenate([lyr['Wf_e'], lyr['Ws_e']], axis=1)
            bias = jnp.concatenate([lyr['bf_e'], lyr['bs_e']])
        ws.append(w.astype(jnp.bfloat16))
        bs.append(bias.reshape(1, -1))

    h = _sc_gather(params['emb_table'], an_pad)[:at]
    e = dist_col
    for li in range(len(layers) - 1):
        nbr_h = _sc_gather(h, nbr_flat)
        h, e = _mp_layer(h, nbr_h, e, mask_col, ws[li], bs[li], fn, nbr)
    nbr_h = _sc_gather(h, nbr_flat)
    forces = _last_layer(
        h, nbr_h, e, mask_col, uv_flat, ws[-1], bs[-1],
        params['out_W1'].astype(jnp.bfloat16), params['out_b1'].reshape(1, -1),
        params['out_W2'].astype(jnp.bfloat16), params['out_b2'].reshape(1, -1),
        fn, nbr)
    return forces.reshape(b, at, 3)


# speedup vs baseline: 5.0714x; 1.0488x over previous
import functools

import jax
import jax.numpy as jnp
from jax import lax
from jax.experimental import pallas as pl
from jax.experimental.pallas import tpu as pltpu
from jax.experimental.pallas import tpu_sc as plsc

GAUSS_END = 5.5

_SC_CORES = 2
_SC_SUBCORES = 16
_NW = _SC_CORES * _SC_SUBCORES
_CHUNK = 128


def _sc_gather(table, idx):
    bsz = idx.shape[0]
    dim = table.shape[1]
    assert bsz % _CHUNK == 0, bsz
    n_chunks = bsz // _CHUNK
    mesh = plsc.VectorSubcoreMesh(core_axis_name="c", subcore_axis_name="s")

    @functools.partial(
        pl.kernel,
        out_type=jax.ShapeDtypeStruct((bsz, dim), table.dtype),
        mesh=mesh,
        scratch_types=[
            pltpu.VMEM((_CHUNK,), jnp.int32),
            pltpu.VMEM((_CHUNK, dim), table.dtype),
            pltpu.SemaphoreType.DMA,
        ],
    )
    def gather_kernel(table_hbm, idx_hbm, out_hbm, idx_v, rows_v, sem):
        wid = lax.axis_index("s") * _SC_CORES + lax.axis_index("c")
        cnt = (n_chunks - wid + _NW - 1) // _NW

        def body(j, carry):
            off = (wid + j * _NW) * _CHUNK
            pltpu.sync_copy(idx_hbm.at[pl.ds(off, _CHUNK)], idx_v)
            pltpu.async_copy(table_hbm.at[idx_v], rows_v, sem).wait()
            pltpu.sync_copy(rows_v, out_hbm.at[pl.ds(off, _CHUNK)])
            return carry

        lax.fori_loop(0, cnt, body, 0)

    return gather_kernel(table, idx)


def _ssp(x):
    return jnp.maximum(x, 0.0) + jnp.log1p(jnp.exp(-jnp.abs(x))) - 0.6931471805599453


def _sigmoid(x):
    return 1.0 / (1.0 + jnp.exp(-x))


_T = 400


def _mp_layer(h, nbr_h, e_in, mask_t, w, b, fn, nbr):
    at = h.shape[0]
    first = e_in.shape[2] == 1
    grid = at // _T
    te = nbr * _T

    def body(h_ref, nbr_ref, e_ref, m_ref, w_ref, b_ref, ho_ref, eo_ref):
        i = pl.program_id(0)
        h_tile = h_ref[pl.ds(i * _T, _T), :]
        if first:
            d = e_ref[...]
            width = GAUSS_END / (fn - 1)
            offs = lax.broadcasted_iota(
                jnp.int32, (1, 1, fn), 2).astype(jnp.float32) * width
            e = jnp.exp(-0.5 * ((d - offs) / width) ** 2)
            eb = e.astype(jnp.bfloat16)
        else:
            eb = e_ref[...]
            e = eb.astype(jnp.float32)
        nbrv = nbr_ref[...]
        z_hi = jnp.dot(h_tile.astype(jnp.bfloat16), w_ref[0:fn, :],
                       preferred_element_type=jnp.float32)
        z = jnp.dot(nbrv.reshape(te, fn).astype(jnp.bfloat16),
                    w_ref[fn:2 * fn, :],
                    preferred_element_type=jnp.float32)
        z = z + jnp.dot(eb.reshape(te, fn), w_ref[2 * fn:3 * fn, :],
                        preferred_element_type=jnp.float32)
        z = z.reshape(nbr, _T, 4 * fn) + z_hi[None] + b_ref[...]
        m = m_ref[...]
        gate_n = _sigmoid(z[:, :, 0:fn])
        core_n = _ssp(z[:, :, fn:2 * fn])
        gate_e = _sigmoid(z[:, :, 2 * fn:3 * fn])
        core_e = _ssp(z[:, :, 3 * fn:4 * fn])
        pn = gate_n * core_n * m
        ho_ref[...] = h_tile + pn.sum(axis=0)
        eo_ref[...] = (e + gate_e * core_e * m).astype(jnp.bfloat16)

    return pl.pallas_call(
        body,
        grid=(grid,),
        in_specs=[
            pl.BlockSpec((at, fn), lambda i: (0, 0)),
            pl.BlockSpec((nbr, _T, fn), lambda i: (0, i, 0)),
            pl.BlockSpec((nbr, _T, e_in.shape[2]), lambda i: (0, i, 0)),
            pl.BlockSpec((nbr, _T, 1), lambda i: (0, i, 0)),
            pl.BlockSpec(w.shape, lambda i: (0, 0)),
            pl.BlockSpec(b.shape, lambda i: (0, 0, 0)),
        ],
        out_specs=[
            pl.BlockSpec((_T, fn), lambda i: (i, 0)),
            pl.BlockSpec((nbr, _T, fn), lambda i: (0, i, 0)),
        ],
        out_shape=[
            jax.ShapeDtypeStruct((at, fn), jnp.float32),
            jax.ShapeDtypeStruct((nbr, at, fn), jnp.bfloat16),
        ],
    )(h, nbr_h, e_in, mask_t, w, b)


def _last_layer(h, nbr_h, e_in, mask_t, uv, w, b, w1, b1, w2, b2, fn, nbr):
    at = h.shape[0]
    grid = at // _T
    te = nbr * _T

    def body(h_ref, nbr_ref, e_ref, m_ref, uv_ref, w_ref, b_ref,
             w1_ref, b1_ref, w2_ref, b2_ref, f_ref):
        i = pl.program_id(0)
        h_tile = h_ref[pl.ds(i * _T, _T), :]
        eb = e_ref[...]
        nbrv = nbr_ref[...]
        z_hi = jnp.dot(h_tile.astype(jnp.bfloat16), w_ref[0:fn, :],
                       preferred_element_type=jnp.float32)
        z = jnp.dot(nbrv.reshape(te, fn).astype(jnp.bfloat16),
                    w_ref[fn:2 * fn, :],
                    preferred_element_type=jnp.float32)
        z = z + jnp.dot(eb.reshape(te, fn), w_ref[2 * fn:3 * fn, :],
                        preferred_element_type=jnp.float32)
        z = z.reshape(nbr, _T, 2 * fn) + z_hi[None] + b_ref[...]
        m = m_ref[...]
        gate_e = _sigmoid(z[:, :, 0:fn])
        core_e = _ssp(z[:, :, fn:2 * fn])
        e3 = eb.astype(jnp.float32) + gate_e * core_e * m
        x = _ssp(jnp.dot(e3.astype(jnp.bfloat16).reshape(te, fn), w1_ref[...],
                         preferred_element_type=jnp.float32) + b1_ref[...])
        s = jnp.dot(x.astype(jnp.bfloat16), w2_ref[...],
                    preferred_element_type=jnp.float32) + b2_ref[...]
        f = s.reshape(nbr, _T, 1) * uv_ref[...] * m
        f_ref[...] = f.sum(axis=0)

    return pl.pallas_call(
        body,
        grid=(grid,),
        in_specs=[
            pl.BlockSpec((at, fn), lambda i: (0, 0)),
            pl.BlockSpec((nbr, _T, fn), lambda i: (0, i, 0)),
            pl.BlockSpec((nbr, _T, fn), lambda i: (0, i, 0)),
            pl.BlockSpec((nbr, _T, 1), lambda i: (0, i, 0)),
            pl.BlockSpec((nbr, _T, 3), lambda i: (0, i, 0)),
            pl.BlockSpec(w.shape, lambda i: (0, 0)),
            pl.BlockSpec(b.shape, lambda i: (0, 0, 0)),
            pl.BlockSpec(w1.shape, lambda i: (0, 0)),
            pl.BlockSpec(b1.shape, lambda i: (0, 0)),
            pl.BlockSpec(w2.shape, lambda i: (0, 0)),
            pl.BlockSpec(b2.shape, lambda i: (0, 0)),
        ],
        out_specs=[pl.BlockSpec((_T, 3), lambda i: (i, 0))],
        out_shape=[jax.ShapeDtypeStruct((at, 3), jnp.float32)],
    )(h, nbr_h, e_in, mask_t, uv, w, b, w1, b1, w2, b2)[0]


def kernel(atomic_numbers, neighbors, neighbor_mask, distances, unit_vecs, params):
    b, at, nbr = neighbors.shape
    fn = params['emb_table'].shape[1]

    an_flat = atomic_numbers.reshape(-1).astype(jnp.int32)
    pad = (-an_flat.shape[0]) % _CHUNK
    an_pad = jnp.pad(an_flat, (0, pad))
    idx_n = jnp.transpose(neighbors.reshape(at, nbr)).reshape(-1).astype(jnp.int32)
    mask_t = jnp.transpose(neighbor_mask.reshape(at, nbr)).reshape(nbr, at, 1)
    dist_t = jnp.transpose(distances.reshape(at, nbr)).reshape(nbr, at, 1)
    uv_t = jnp.transpose(unit_vecs.reshape(at, nbr, 3), (1, 0, 2))

    layers = params['layers']
    ws, bs = [], []
    for li in range(len(layers)):
        lyr = layers[li]
        if li + 1 < len(layers):
            w = jnp.concatenate(
                [lyr['Wf_n'], lyr['Ws_n'], lyr['Wf_e'], lyr['Ws_e']], axis=1)
            bias = jnp.concatenate(
                [lyr['bf_n'], lyr['bs_n'], lyr['bf_e'], lyr['bs_e']])
        else:
            w = jnp.concatenate([lyr['Wf_e'], lyr['Ws_e']], axis=1)
            bias = jnp.concatenate([lyr['bf_e'], lyr['bs_e']])
        ws.append(w.astype(jnp.bfloat16))
        bs.append(bias.reshape(1, 1, -1))

    h = _sc_gather(params['emb_table'], an_pad)[:at]
    e = dist_t
    for li in range(len(layers) - 1):
        nbr_h = _sc_gather(h, idx_n).reshape(nbr, at, fn)
        h, e = _mp_layer(h, nbr_h, e, mask_t, ws[li], bs[li], fn, nbr)
    nbr_h = _sc_gather(h, idx_n).reshape(nbr, at, fn)
    forces = _last_layer(
        h, nbr_h, e, mask_t, uv_t, ws[-1], bs[-1],
        params['out_W1'].astype(jnp.bfloat16), params['out_b1'].reshape(1, -1),
        params['out_W2'].astype(jnp.bfloat16), params['out_b2'].reshape(1, -1),
        fn, nbr)
    return forces.reshape(b, at, 3)


# speedup vs baseline: 5.9614x; 1.1755x over previous
import functools

import jax
import jax.numpy as jnp
from jax import lax
from jax.experimental import pallas as pl
from jax.experimental.pallas import tpu as pltpu
from jax.experimental.pallas import tpu_sc as plsc

GAUSS_END = 5.5

_SC_CORES = 2
_SC_SUBCORES = 16
_NW = _SC_CORES * _SC_SUBCORES
_CHUNK = 128


def _sc_gather(table, idx):
    bsz = idx.shape[0]
    dim = table.shape[1]
    assert bsz % _CHUNK == 0, bsz
    n_chunks = bsz // _CHUNK
    mesh = plsc.VectorSubcoreMesh(core_axis_name="c", subcore_axis_name="s")

    @functools.partial(
        pl.kernel,
        out_type=jax.ShapeDtypeStruct((bsz, dim), table.dtype),
        mesh=mesh,
        scratch_types=[
            pltpu.VMEM((_CHUNK,), jnp.int32),
            pltpu.VMEM((_CHUNK, dim), table.dtype),
            pltpu.SemaphoreType.DMA,
        ],
    )
    def gather_kernel(table_hbm, idx_hbm, out_hbm, idx_v, rows_v, sem):
        wid = lax.axis_index("s") * _SC_CORES + lax.axis_index("c")
        cnt = (n_chunks - wid + _NW - 1) // _NW

        def body(j, carry):
            off = (wid + j * _NW) * _CHUNK
            pltpu.sync_copy(idx_hbm.at[pl.ds(off, _CHUNK)], idx_v)
            pltpu.async_copy(table_hbm.at[idx_v], rows_v, sem).wait()
            pltpu.sync_copy(rows_v, out_hbm.at[pl.ds(off, _CHUNK)])
            return carry

        lax.fori_loop(0, cnt, body, 0)

    return gather_kernel(table, idx)


_LOG2E = 1.4426950408889634
_LN2 = 0.6931471805599453


def _neg_abs(x):
    xi = lax.bitcast_convert_type(x, jnp.int32)
    return lax.bitcast_convert_type(
        jnp.bitwise_or(xi, jnp.int32(-2147483648)), jnp.float32)


def _gate(v):
    return 1.0 / (1.0 + jnp.exp2(v))


def _core(u):
    return jnp.maximum(u, 0.0) + jnp.log2(1.0 + jnp.exp2(_neg_abs(u))) - 1.0


_T = 400


def _mp_layer(h, nbr_h, e_in, mask_t, w, b, fn, nbr):
    at = h.shape[0]
    first = e_in.shape[2] == 1
    grid = at // _T
    te = nbr * _T

    def body(h_ref, nbr_ref, e_ref, m_ref, w_ref, b_ref, ho_ref, eo_ref):
        i = pl.program_id(0)
        h_tile = h_ref[pl.ds(i * _T, _T), :]
        if first:
            d = e_ref[...]
            inv_w = (fn - 1) / GAUSS_END
            offs = lax.broadcasted_iota(
                jnp.int32, (1, 1, fn), 2).astype(jnp.float32)
            t = d * inv_w - offs
            e = jnp.exp2((-0.5 * _LOG2E) * t * t)
            eb = e.astype(jnp.bfloat16)
        else:
            eb = e_ref[...]
            e = eb.astype(jnp.float32)
        nbrv = nbr_ref[...]
        z_hi = jnp.dot(h_tile.astype(jnp.bfloat16), w_ref[0:fn, :],
                       preferred_element_type=jnp.float32) + b_ref[0]
        z = jnp.dot(nbrv.reshape(te, fn).astype(jnp.bfloat16),
                    w_ref[fn:2 * fn, :],
                    preferred_element_type=jnp.float32)
        z = z + jnp.dot(eb.reshape(te, fn), w_ref[2 * fn:3 * fn, :],
                        preferred_element_type=jnp.float32)
        z = z.reshape(nbr, _T, 4 * fn) + z_hi[None]
        m = m_ref[...] * _LN2
        gate_n = _gate(z[:, :, 0:fn])
        core_n = _core(z[:, :, fn:2 * fn])
        gate_e = _gate(z[:, :, 2 * fn:3 * fn])
        core_e = _core(z[:, :, 3 * fn:4 * fn])
        pn = gate_n * core_n * m
        ho_ref[...] = h_tile + pn.sum(axis=0)
        eo_ref[...] = (e + gate_e * core_e * m).astype(jnp.bfloat16)

    return pl.pallas_call(
        body,
        grid=(grid,),
        in_specs=[
            pl.BlockSpec((at, fn), lambda i: (0, 0)),
            pl.BlockSpec((nbr, _T, fn), lambda i: (0, i, 0)),
            pl.BlockSpec((nbr, _T, e_in.shape[2]), lambda i: (0, i, 0)),
            pl.BlockSpec((nbr, _T, 1), lambda i: (0, i, 0)),
            pl.BlockSpec(w.shape, lambda i: (0, 0)),
            pl.BlockSpec(b.shape, lambda i: (0, 0)),
        ],
        out_specs=[
            pl.BlockSpec((_T, fn), lambda i: (i, 0)),
            pl.BlockSpec((nbr, _T, fn), lambda i: (0, i, 0)),
        ],
        out_shape=[
            jax.ShapeDtypeStruct((at, fn), jnp.float32),
            jax.ShapeDtypeStruct((nbr, at, fn), jnp.bfloat16),
        ],
    )(h, nbr_h, e_in, mask_t, w, b)


def _last_layer(h, nbr_h, e_in, mask_t, uv, w, b, w1, b1, w2, b2, fn, nbr):
    at = h.shape[0]
    grid = at // _T
    te = nbr * _T

    def body(h_ref, nbr_ref, e_ref, m_ref, uv_ref, w_ref, b_ref,
             w1_ref, b1_ref, w2_ref, b2_ref, f_ref):
        i = pl.program_id(0)
        h_tile = h_ref[pl.ds(i * _T, _T), :]
        eb = e_ref[...]
        nbrv = nbr_ref[...]
        z_hi = jnp.dot(h_tile.astype(jnp.bfloat16), w_ref[0:fn, :],
                       preferred_element_type=jnp.float32) + b_ref[0]
        z = jnp.dot(nbrv.reshape(te, fn).astype(jnp.bfloat16),
                    w_ref[fn:2 * fn, :],
                    preferred_element_type=jnp.float32)
        z = z + jnp.dot(eb.reshape(te, fn), w_ref[2 * fn:3 * fn, :],
                        preferred_element_type=jnp.float32)
        z = z.reshape(nbr, _T, 2 * fn) + z_hi[None]
        m = m_ref[...]
        m2 = m * _LN2
        gate_e = _gate(z[:, :, 0:fn])
        core_e = _core(z[:, :, fn:2 * fn])
        e3 = eb.astype(jnp.float32) + gate_e * core_e * m2
        x = _core(jnp.dot(e3.astype(jnp.bfloat16).reshape(te, fn), w1_ref[...],
                          preferred_element_type=jnp.float32) + b1_ref[...])
        s = jnp.dot(x.astype(jnp.bfloat16), w2_ref[...],
                    preferred_element_type=jnp.float32) + b2_ref[...]
        f = s.reshape(nbr, _T, 1) * uv_ref[...] * m
        f_ref[...] = f.sum(axis=0)

    return pl.pallas_call(
        body,
        grid=(grid,),
        in_specs=[
            pl.BlockSpec((at, fn), lambda i: (0, 0)),
            pl.BlockSpec((nbr, _T, fn), lambda i: (0, i, 0)),
            pl.BlockSpec((nbr, _T, fn), lambda i: (0, i, 0)),
            pl.BlockSpec((nbr, _T, 1), lambda i: (0, i, 0)),
            pl.BlockSpec((nbr, _T, 3), lambda i: (0, i, 0)),
            pl.BlockSpec(w.shape, lambda i: (0, 0)),
            pl.BlockSpec(b.shape, lambda i: (0, 0)),
            pl.BlockSpec(w1.shape, lambda i: (0, 0)),
            pl.BlockSpec(b1.shape, lambda i: (0, 0)),
            pl.BlockSpec(w2.shape, lambda i: (0, 0)),
            pl.BlockSpec(b2.shape, lambda i: (0, 0)),
        ],
        out_specs=[pl.BlockSpec((_T, 3), lambda i: (i, 0))],
        out_shape=[jax.ShapeDtypeStruct((at, 3), jnp.float32)],
    )(h, nbr_h, e_in, mask_t, uv, w, b, w1, b1, w2, b2)[0]


def kernel(atomic_numbers, neighbors, neighbor_mask, distances, unit_vecs, params):
    b, at, nbr = neighbors.shape
    fn = params['emb_table'].shape[1]

    an_flat = atomic_numbers.reshape(-1).astype(jnp.int32)
    pad = (-an_flat.shape[0]) % _CHUNK
    an_pad = jnp.pad(an_flat, (0, pad))
    idx_n = jnp.transpose(neighbors.reshape(at, nbr)).reshape(-1).astype(jnp.int32)
    mask_t = jnp.transpose(neighbor_mask.reshape(at, nbr)).reshape(nbr, at, 1)
    dist_t = jnp.transpose(distances.reshape(at, nbr)).reshape(nbr, at, 1)
    uv_t = jnp.transpose(unit_vecs.reshape(at, nbr, 3), (1, 0, 2))

    layers = params['layers']
    ws, bs = [], []
    gl, cl = -_LOG2E, _LOG2E
    for li in range(len(layers)):
        lyr = layers[li]
        if li + 1 < len(layers):
            w = jnp.concatenate(
                [gl * lyr['Wf_n'], cl * lyr['Ws_n'],
                 gl * lyr['Wf_e'], cl * lyr['Ws_e']], axis=1)
            bias = jnp.concatenate(
                [gl * lyr['bf_n'], cl * lyr['bs_n'],
                 gl * lyr['bf_e'], cl * lyr['bs_e']])
        else:
            w = jnp.concatenate([gl * lyr['Wf_e'], cl * lyr['Ws_e']], axis=1)
            bias = jnp.concatenate([gl * lyr['bf_e'], cl * lyr['bs_e']])
        ws.append(w.astype(jnp.bfloat16))
        bs.append(bias.reshape(1, -1))

    h = _sc_gather(params['emb_table'], an_pad)[:at]
    e = dist_t
    for li in range(len(layers) - 1):
        nbr_h = _sc_gather(h, idx_n).reshape(nbr, at, fn)
        h, e = _mp_layer(h, nbr_h, e, mask_t, ws[li], bs[li], fn, nbr)
    nbr_h = _sc_gather(h, idx_n).reshape(nbr, at, fn)
    forces = _last_layer(
        h, nbr_h, e, mask_t, uv_t, ws[-1], bs[-1],
        (_LOG2E * params['out_W1']).astype(jnp.bfloat16),
        _LOG2E * params['out_b1'].reshape(1, -1),
        (_LN2 * params['out_W2']).astype(jnp.bfloat16),
        params['out_b2'].reshape(1, -1),
        fn, nbr)
    return forces.reshape(b, at, 3)


# speedup vs baseline: 6.5187x; 1.0935x over previous
import functools

import jax
import jax.numpy as jnp
from jax import lax
from jax.experimental import pallas as pl
from jax.experimental.pallas import tpu as pltpu
from jax.experimental.pallas import tpu_sc as plsc

GAUSS_END = 5.5

_SC_CORES = 2
_SC_SUBCORES = 16
_NW = _SC_CORES * _SC_SUBCORES
_CHUNK = 128


def _sc_gather(table, idx):
    bsz = idx.shape[0]
    dim = table.shape[1]
    assert bsz % (8 * _NW) == 0, bsz
    bpw = bsz // _NW
    full = bpw // _CHUNK
    rem = bpw % _CHUNK
    assert rem % 8 == 0, rem
    full_p = full if full % 2 == 1 else full - 1
    mesh = plsc.VectorSubcoreMesh(core_axis_name="c", subcore_axis_name="s")

    @functools.partial(
        pl.kernel,
        out_type=jax.ShapeDtypeStruct((bsz, dim), table.dtype),
        mesh=mesh,
        scratch_types=[
            pltpu.VMEM((bpw,), jnp.int32),
            pltpu.VMEM((_CHUNK, dim), table.dtype),
            pltpu.VMEM((_CHUNK, dim), table.dtype),
            pltpu.SemaphoreType.DMA,
            pltpu.SemaphoreType.DMA,
        ],
    )
    def gather_kernel(table_hbm, idx_hbm, out_hbm, idx_v, r0, r1, s0, s1):
        wid = lax.axis_index("s") * _SC_CORES + lax.axis_index("c")
        base = wid * bpw
        pltpu.sync_copy(idx_hbm.at[pl.ds(base, bpw)], idx_v)

        def start(c, rv, sem):
            iv = idx_v.at[pl.ds(c * _CHUNK, _CHUNK)]
            pltpu.async_copy(table_hbm.at[iv], rv, sem)

        def drain(c, rv, sem):
            pltpu.make_async_copy(table_hbm.at[idx_v.at[pl.ds(0, _CHUNK)]],
                                  rv, sem).wait()
            pltpu.sync_copy(rv, out_hbm.at[pl.ds(base + c * _CHUNK, _CHUNK)])

        if full_p >= 1:
            start(0, r0, s0)

            def pair(p, carry):
                start(2 * p + 1, r1, s1)
                drain(2 * p, r0, s0)
                start(2 * p + 2, r0, s0)
                drain(2 * p + 1, r1, s1)
                return carry

            lax.fori_loop(0, (full_p - 1) // 2, pair, 0)
            drain(full_p - 1, r0, s0)
        if full_p < full:
            c = full_p
            start(c, r1, s1)
            drain(c, r1, s1)
        if rem:
            iv = idx_v.at[pl.ds(full * _CHUNK, rem)]
            rv = r0.at[pl.ds(0, rem)]
            pltpu.async_copy(table_hbm.at[iv], rv, s0).wait()
            pltpu.sync_copy(rv, out_hbm.at[pl.ds(base + full * _CHUNK, rem)])

    return gather_kernel(table, idx)


_LOG2E = 1.4426950408889634
_LN2 = 0.6931471805599453


def _neg_abs(x):
    xi = lax.bitcast_convert_type(x, jnp.int32)
    return lax.bitcast_convert_type(
        jnp.bitwise_or(xi, jnp.int32(-2147483648)), jnp.float32)


def _gate(v):
    return 1.0 / (1.0 + jnp.exp2(v))


def _core(u):
    return jnp.maximum(u, 0.0) + jnp.log2(1.0 + jnp.exp2(_neg_abs(u))) - 1.0


_T = 400


def _mp_layer(h, nbr_h, e_in, mask_t, w, b, fn, nbr):
    at = h.shape[0]
    first = e_in.shape[2] == 1
    grid = at // _T
    te = nbr * _T

    def body(h_ref, nbr_ref, e_ref, m_ref, w_ref, b_ref, ho_ref, eo_ref):
        i = pl.program_id(0)
        h_tile = h_ref[pl.ds(i * _T, _T), :]
        if first:
            d = e_ref[...]
            inv_w = (fn - 1) / GAUSS_END
            offs = lax.broadcasted_iota(
                jnp.int32, (1, 1, fn), 2).astype(jnp.float32)
            t = d * inv_w - offs
            e = jnp.exp2((-0.5 * _LOG2E) * t * t)
            eb = e.astype(jnp.bfloat16)
        else:
            eb = e_ref[...]
            e = eb.astype(jnp.float32)
        nbrv = nbr_ref[...]
        z_hi = jnp.dot(h_tile.astype(jnp.bfloat16), w_ref[0:fn, :],
                       preferred_element_type=jnp.float32) + b_ref[0]
        z = jnp.dot(nbrv.reshape(te, fn).astype(jnp.bfloat16),
                    w_ref[fn:2 * fn, :],
                    preferred_element_type=jnp.float32)
        z = z + jnp.dot(eb.reshape(te, fn), w_ref[2 * fn:3 * fn, :],
                        preferred_element_type=jnp.float32)
        z = z.reshape(nbr, _T, 4 * fn) + z_hi[None]
        m = m_ref[...] * _LN2
        gate_n = _gate(z[:, :, 0:fn])
        core_n = _core(z[:, :, fn:2 * fn])
        gate_e = _gate(z[:, :, 2 * fn:3 * fn])
        core_e = _core(z[:, :, 3 * fn:4 * fn])
        pn = gate_n * core_n * m
        ho_ref[...] = h_tile + pn.sum(axis=0)
        eo_ref[...] = (e + gate_e * core_e * m).astype(jnp.bfloat16)

    return pl.pallas_call(
        body,
        grid=(grid,),
        in_specs=[
            pl.BlockSpec((at, fn), lambda i: (0, 0)),
            pl.BlockSpec((nbr, _T, fn), lambda i: (0, i, 0)),
            pl.BlockSpec((nbr, _T, e_in.shape[2]), lambda i: (0, i, 0)),
            pl.BlockSpec((nbr, _T, 1), lambda i: (0, i, 0)),
            pl.BlockSpec(w.shape, lambda i: (0, 0)),
            pl.BlockSpec(b.shape, lambda i: (0, 0)),
        ],
        out_specs=[
            pl.BlockSpec((_T, fn), lambda i: (i, 0)),
            pl.BlockSpec((nbr, _T, fn), lambda i: (0, i, 0)),
        ],
        out_shape=[
            jax.ShapeDtypeStruct((at, fn), jnp.float32),
            jax.ShapeDtypeStruct((nbr, at, fn), jnp.bfloat16),
        ],
    )(h, nbr_h, e_in, mask_t, w, b)


def _last_layer(h, nbr_h, e_in, mask_t, uv, w, b, w1, b1, w2, b2, fn, nbr):
    at = h.shape[0]
    grid = at // _T
    te = nbr * _T

    def body(h_ref, nbr_ref, e_ref, m_ref, uv_ref, w_ref, b_ref,
             w1_ref, b1_ref, w2_ref, b2_ref, f_ref):
        i = pl.program_id(0)
        h_tile = h_ref[pl.ds(i * _T, _T), :]
        eb = e_ref[...]
        nbrv = nbr_ref[...]
        z_hi = jnp.dot(h_tile.astype(jnp.bfloat16), w_ref[0:fn, :],
                       preferred_element_type=jnp.float32) + b_ref[0]
        z = jnp.dot(nbrv.reshape(te, fn).astype(jnp.bfloat16),
                    w_ref[fn:2 * fn, :],
                    preferred_element_type=jnp.float32)
        z = z + jnp.dot(eb.reshape(te, fn), w_ref[2 * fn:3 * fn, :],
                        preferred_element_type=jnp.float32)
        z = z.reshape(nbr, _T, 2 * fn) + z_hi[None]
        m = m_ref[...]
        m2 = m * _LN2
        gate_e = _gate(z[:, :, 0:fn])
        core_e = _core(z[:, :, fn:2 * fn])
        e3 = eb.astype(jnp.float32) + gate_e * core_e * m2
        x = _core(jnp.dot(e3.astype(jnp.bfloat16).reshape(te, fn), w1_ref[...],
                          preferred_element_type=jnp.float32) + b1_ref[...])
        s = jnp.dot(x.astype(jnp.bfloat16), w2_ref[...],
                    preferred_element_type=jnp.float32) + b2_ref[...]
        f = s.reshape(nbr, _T, 1) * uv_ref[...] * m
        f_ref[...] = f.sum(axis=0)

    return pl.pallas_call(
        body,
        grid=(grid,),
        in_specs=[
            pl.BlockSpec((at, fn), lambda i: (0, 0)),
            pl.BlockSpec((nbr, _T, fn), lambda i: (0, i, 0)),
            pl.BlockSpec((nbr, _T, fn), lambda i: (0, i, 0)),
            pl.BlockSpec((nbr, _T, 1), lambda i: (0, i, 0)),
            pl.BlockSpec((nbr, _T, 3), lambda i: (0, i, 0)),
            pl.BlockSpec(w.shape, lambda i: (0, 0)),
            pl.BlockSpec(b.shape, lambda i: (0, 0)),
            pl.BlockSpec(w1.shape, lambda i: (0, 0)),
            pl.BlockSpec(b1.shape, lambda i: (0, 0)),
            pl.BlockSpec(w2.shape, lambda i: (0, 0)),
            pl.BlockSpec(b2.shape, lambda i: (0, 0)),
        ],
        out_specs=[pl.BlockSpec((_T, 3), lambda i: (i, 0))],
        out_shape=[jax.ShapeDtypeStruct((at, 3), jnp.float32)],
    )(h, nbr_h, e_in, mask_t, uv, w, b, w1, b1, w2, b2)[0]


def kernel(atomic_numbers, neighbors, neighbor_mask, distances, unit_vecs, params):
    b, at, nbr = neighbors.shape
    fn = params['emb_table'].shape[1]

    an_flat = atomic_numbers.reshape(-1).astype(jnp.int32)
    pad = (-an_flat.shape[0]) % (8 * _NW)
    an_pad = jnp.pad(an_flat, (0, pad))
    idx_n = jnp.transpose(neighbors.reshape(at, nbr)).reshape(-1).astype(jnp.int32)
    mask_t = jnp.transpose(neighbor_mask.reshape(at, nbr)).reshape(nbr, at, 1)
    dist_t = jnp.transpose(distances.reshape(at, nbr)).reshape(nbr, at, 1)
    uv_t = jnp.transpose(unit_vecs.reshape(at, nbr, 3), (1, 0, 2))

    layers = params['layers']
    ws, bs = [], []
    gl, cl = -_LOG2E, _LOG2E
    for li in range(len(layers)):
        lyr = layers[li]
        if li + 1 < len(layers):
            w = jnp.concatenate(
                [gl * lyr['Wf_n'], cl * lyr['Ws_n'],
                 gl * lyr['Wf_e'], cl * lyr['Ws_e']], axis=1)
            bias = jnp.concatenate(
                [gl * lyr['bf_n'], cl * lyr['bs_n'],
                 gl * lyr['bf_e'], cl * lyr['bs_e']])
        else:
            w = jnp.concatenate([gl * lyr['Wf_e'], cl * lyr['Ws_e']], axis=1)
            bias = jnp.concatenate([gl * lyr['bf_e'], cl * lyr['bs_e']])
        ws.append(w.astype(jnp.bfloat16))
        bs.append(bias.reshape(1, -1))

    h = _sc_gather(params['emb_table'], an_pad)[:at]
    e = dist_t
    for li in range(len(layers) - 1):
        nbr_h = _sc_gather(h, idx_n).reshape(nbr, at, fn)
        h, e = _mp_layer(h, nbr_h, e, mask_t, ws[li], bs[li], fn, nbr)
    nbr_h = _sc_gather(h, idx_n).reshape(nbr, at, fn)
    forces = _last_layer(
        h, nbr_h, e, mask_t, uv_t, ws[-1], bs[-1],
        (_LOG2E * params['out_W1']).astype(jnp.bfloat16),
        _LOG2E * params['out_b1'].reshape(1, -1),
        (_LN2 * params['out_W2']).astype(jnp.bfloat16),
        params['out_b2'].reshape(1, -1),
        fn, nbr)
    return forces.reshape(b, at, 3)
